# Initial kernel scaffold; baseline (speedup 1.0000x reference)
import functools

import jax
import jax.numpy as jnp
from jax import lax
from jax.experimental import pallas as pl
from jax.experimental.pallas import tpu as pltpu
from jax.experimental.pallas import tpu_sc as plsc

NC = 2
NS = 16
NW = NC * NS
L = 16
CH = 128

_ZEROS16 = jnp.zeros((L,), jnp.float32)


def _sc_mesh():
    return plsc.VectorSubcoreMesh(core_axis_name="c", subcore_axis_name="s")


def _make_deg_kernel(e_pad, ndeg):
    chunks_per_tile = e_pad // (NW * CH)
    rows_per_tile = ndeg // NS

    @functools.partial(
        pl.kernel,
        out_type=jax.ShapeDtypeStruct((NC, ndeg), jnp.float32),
        mesh=_sc_mesh(),
        scratch_types=[
            pltpu.VMEM((CH,), jnp.int32),
            pltpu.VMEM((CH,), jnp.float32),
            pltpu.VMEM((rows_per_tile,), jnp.float32),
            pltpu.VMEM_SHARED((ndeg,), jnp.float32),
        ],
    )
    def deg_kernel(col_hbm, deg_out, colbuf, ones, zbuf, deg_sp):
        cidx = lax.axis_index("c")
        sidx = lax.axis_index("s")
        wid = sidx * NC + cidx

        def zinit(i, _):
            zbuf[pl.ds(i * L, L)] = _ZEROS16
            return 0
        lax.fori_loop(0, rows_per_tile // L, zinit, 0)
        for k in range(CH // L):
            ones[pl.ds(k * L, L)] = jnp.full((L,), 1.0, jnp.float32)
        pltpu.sync_copy(zbuf, deg_sp.at[pl.ds(sidx * rows_per_tile, rows_per_tile)])
        plsc.subcore_barrier()

        def body(i, _):
            base = (wid * chunks_per_tile + i) * CH
            pltpu.sync_copy(col_hbm.at[pl.ds(base, CH)], colbuf)
            pltpu.sync_copy(ones, deg_sp.at[colbuf], add=True)
            return 0
        lax.fori_loop(0, chunks_per_tile, body, 0)

        plsc.subcore_barrier()
        pltpu.sync_copy(
            deg_sp.at[pl.ds(sidx * rows_per_tile, rows_per_tile)],
            deg_out.at[cidx, pl.ds(sidx * rows_per_tile, rows_per_tile)],
        )

    return deg_kernel


def _make_agg_kernel(e_pad, n_pad, d):
    chunks_per_tile = e_pad // (NW * CH)
    rows_per_tile = n_pad // NS

    @functools.partial(
        pl.kernel,
        out_type=jax.ShapeDtypeStruct((NC, n_pad, d), jnp.float32),
        mesh=_sc_mesh(),
        scratch_types=[
            pltpu.VMEM((CH,), jnp.int32),
            pltpu.VMEM((CH,), jnp.int32),
            pltpu.VMEM((CH, d), jnp.float32),
            pltpu.SemaphoreType.DMA,
            pltpu.VMEM_SHARED((n_pad, d), jnp.float32),
        ],
    )
    def agg_kernel(xp_hbm, row_hbm, col_hbm, p_out,
                   rowbuf, colbuf, gbuf, sem, y_sp):
        cidx = lax.axis_index("c")
        sidx = lax.axis_index("s")
        wid = sidx * NC + cidx

        def zinit(t, _):
            r = t // (d // L)
            k = t % (d // L)
            gbuf[r, pl.ds(k * L, L)] = _ZEROS16
            return 0
        lax.fori_loop(0, CH * (d // L), zinit, 0)

        base_row = sidx * rows_per_tile
        full = rows_per_tile // CH
        rem = rows_per_tile - full * CH
        for j in range(full):
            pltpu.sync_copy(gbuf, y_sp.at[pl.ds(base_row + j * CH, CH)])
        if rem:
            pltpu.sync_copy(gbuf.at[pl.ds(0, rem)],
                            y_sp.at[pl.ds(base_row + full * CH, rem)])
        plsc.subcore_barrier()

        def body(i, _):
            base = (wid * chunks_per_tile + i) * CH
            pltpu.sync_copy(row_hbm.at[pl.ds(base, CH)], rowbuf)
            pltpu.sync_copy(col_hbm.at[pl.ds(base, CH)], colbuf)
            pltpu.async_copy(xp_hbm.at[rowbuf], gbuf, sem).wait()
            pltpu.sync_copy(gbuf, y_sp.at[colbuf], add=True)
            return 0
        lax.fori_loop(0, chunks_per_tile, body, 0)

        plsc.subcore_barrier()
        pltpu.sync_copy(
            y_sp.at[pl.ds(base_row, rows_per_tile)],
            p_out.at[cidx, pl.ds(base_row, rows_per_tile)],
        )

    return agg_kernel


def _prep_body(degp_ref, x_ref, xp_ref, dinv_ref):
    p = degp_ref[...]
    deg = p[0] + p[1] + 1.0
    dinv = lax.rsqrt(deg)
    dinv_ref[...] = dinv
    xp_ref[...] = x_ref[...] * dinv


def _prep_call(degp3, x, blk):
    n, d = x.shape
    grid = n // blk
    return pl.pallas_call(
        _prep_body,
        grid=(grid,),
        in_specs=[
            pl.BlockSpec((NC, blk, 1), lambda i: (0, i, 0)),
            pl.BlockSpec((blk, d), lambda i: (i, 0)),
        ],
        out_specs=[
            pl.BlockSpec((blk, d), lambda i: (i, 0)),
            pl.BlockSpec((blk, 1), lambda i: (i, 0)),
        ],
        out_shape=[
            jax.ShapeDtypeStruct((n, d), jnp.float32),
            jax.ShapeDtypeStruct((n, 1), jnp.float32),
        ],
    )(degp3, x)


def _dense_body(p_ref, xp_ref, dinv_ref, wh_ref, bh_ref, wt_ref, bt_ref,
                wf_ref, bf_ref, wc_ref, bc_ref, out_ref):
    p = p_ref[...]
    y = dinv_ref[...] * (p[0] + p[1] + xp_ref[...])
    hh = jnp.maximum(jnp.dot(y, wh_ref[...], preferred_element_type=jnp.float32)
                     + bh_ref[...], 0.0)
    ht = jnp.maximum(jnp.dot(y, wt_ref[...], preferred_element_type=jnp.float32)
                     + bt_ref[...], 0.0)
    hf = 0.5 * hh + 0.5 * ht
    h = jnp.maximum(jnp.dot(hf, wf_ref[...], preferred_element_type=jnp.float32)
                    + bf_ref[...], 0.0)
    out_ref[...] = jnp.dot(h, wc_ref[...], preferred_element_type=jnp.float32) \
        + bc_ref[...]


def _dense_call(p, xp, dinv, wh, bh, wt, bt, wf, bf, wc_pad, bc_pad, blk):
    n, d = xp.shape
    hid = wh.shape[1]
    f = wf.shape[1]
    cpad = wc_pad.shape[1]
    grid = n // blk

    def full(shape):
        return pl.BlockSpec(shape, lambda i: tuple(0 for _ in shape))

    return pl.pallas_call(
        _dense_body,
        grid=(grid,),
        in_specs=[
            pl.BlockSpec((NC, blk, d), lambda i: (0, i, 0)),
            pl.BlockSpec((blk, d), lambda i: (i, 0)),
            pl.BlockSpec((blk, 1), lambda i: (i, 0)),
            full((d, hid)), full((1, hid)),
            full((d, hid)), full((1, hid)),
            full((hid, f)), full((1, f)),
            full((f, cpad)), full((1, cpad)),
        ],
        out_specs=pl.BlockSpec((blk, cpad), lambda i: (i, 0)),
        out_shape=jax.ShapeDtypeStruct((n, cpad), jnp.float32),
    )(p, xp, dinv, wh, bh, wt, bt, wf, bf, wc_pad, bc_pad)


def kernel(x, edge_index, h_node, W_homo, b_homo, W_hetero, b_hetero,
           W_fus, b_fus, W_cls, b_cls):
    del h_node
    n, d = x.shape
    e = edge_index.shape[1]

    n_pad = ((n + L) // L) * L
    ndeg = ((n + 1 + NS * L - 1) // (NS * L)) * (NS * L)
    e_pad = ((e + NW * CH - 1) // (NW * CH)) * (NW * CH)

    row = edge_index[0].astype(jnp.int32)
    col = edge_index[1].astype(jnp.int32)
    pad = jnp.full((e_pad - e,), n, jnp.int32)
    row_pad = jnp.concatenate([row, pad])
    col_pad = jnp.concatenate([col, pad])

    degp = _make_deg_kernel(e_pad, ndeg)(col_pad)
    degp3 = degp[:, :n].reshape(NC, n, 1)

    blk = 2000
    xprime, dinv = _prep_call(degp3, x, blk)

    xp_pad = jnp.pad(xprime, ((0, n_pad - n), (0, 0)))
    p = _make_agg_kernel(e_pad, n_pad, d)(xp_pad, row_pad, col_pad)

    bh = b_homo.reshape(1, -1)
    bt = b_hetero.reshape(1, -1)
    bf = b_fus.reshape(1, -1)
    ncls = W_cls.shape[1]
    wc_pad = jnp.pad(W_cls, ((0, 0), (0, 128 - ncls)))
    bc_pad = jnp.pad(b_cls.reshape(1, -1), ((0, 0), (0, 128 - ncls)))

    logits128 = _dense_call(p, xprime, dinv, W_homo, bh, W_hetero, bt,
                            W_fus, bf, wc_pad, bc_pad, blk)
    return logits128[:, :ncls]


# speedup vs baseline: 19.2518x; 19.2518x over previous
import functools

import jax
import jax.numpy as jnp
from jax import lax
from jax.experimental import pallas as pl
from jax.experimental.pallas import tpu as pltpu
from jax.experimental.pallas import tpu_sc as plsc

NC = 2
NS = 16
NW = NC * NS
L = 16
CH = 128


def _sc_mesh():
    return plsc.VectorSubcoreMesh(core_axis_name="c", subcore_axis_name="s")


def _make_deg_kernel(e_pad, ndeg):
    chunks_per_tile = e_pad // (NW * CH)
    rows_per_tile = ndeg // NS

    @functools.partial(
        pl.kernel,
        out_type=jax.ShapeDtypeStruct((NC, ndeg), jnp.float32),
        mesh=_sc_mesh(),
        scratch_types=[
            pltpu.VMEM((CH,), jnp.int32),
            pltpu.VMEM((CH,), jnp.float32),
            pltpu.VMEM((rows_per_tile,), jnp.float32),
            pltpu.VMEM_SHARED((ndeg,), jnp.float32),
        ],
    )
    def deg_kernel(col_hbm, deg_out, colbuf, ones, zbuf, deg_sp):
        cidx = lax.axis_index("c")
        sidx = lax.axis_index("s")
        wid = sidx * NC + cidx

        def zinit(i, _):
            zbuf[pl.ds(i * L, L)] = jnp.zeros((L,), jnp.float32)
            return 0
        lax.fori_loop(0, rows_per_tile // L, zinit, 0)
        for k in range(CH // L):
            ones[pl.ds(k * L, L)] = jnp.full((L,), 1.0, jnp.float32)
        pltpu.sync_copy(zbuf, deg_sp.at[pl.ds(sidx * rows_per_tile, rows_per_tile)])
        plsc.subcore_barrier()

        def body(i, _):
            base = (wid * chunks_per_tile + i) * CH
            pltpu.sync_copy(col_hbm.at[pl.ds(base, CH)], colbuf)
            pltpu.sync_copy(ones, deg_sp.at[colbuf], add=True)
            return 0
        lax.fori_loop(0, chunks_per_tile, body, 0)

        plsc.subcore_barrier()
        pltpu.sync_copy(
            deg_sp.at[pl.ds(sidx * rows_per_tile, rows_per_tile)],
            deg_out.at[cidx, pl.ds(sidx * rows_per_tile, rows_per_tile)],
        )

    return deg_kernel


def _make_agg_kernel(e_pad, n_pad, d):
    chunks_per_tile = e_pad // (NW * CH)
    rows_per_tile = n_pad // NS

    @functools.partial(
        pl.kernel,
        out_type=jax.ShapeDtypeStruct((NC, n_pad, d), jnp.float32),
        mesh=_sc_mesh(),
        scratch_types=[
            pltpu.VMEM((CH,), jnp.int32),
            pltpu.VMEM((CH,), jnp.int32),
            pltpu.VMEM((CH, d), jnp.float32),
            pltpu.SemaphoreType.DMA,
            pltpu.VMEM_SHARED((n_pad, d), jnp.float32),
        ],
    )
    def agg_kernel(xp_hbm, row_hbm, col_hbm, p_out,
                   rowbuf, colbuf, gbuf, sem, y_sp):
        cidx = lax.axis_index("c")
        sidx = lax.axis_index("s")
        wid = sidx * NC + cidx

        def zinit(t, _):
            r = t // (d // L)
            k = t % (d // L)
            gbuf[r, pl.ds(k * L, L)] = jnp.zeros((L,), jnp.float32)
            return 0
        lax.fori_loop(0, CH * (d // L), zinit, 0)

        base_row = sidx * rows_per_tile
        full = rows_per_tile // CH
        rem = rows_per_tile - full * CH
        for j in range(full):
            pltpu.sync_copy(gbuf, y_sp.at[pl.ds(base_row + j * CH, CH)])
        if rem:
            pltpu.sync_copy(gbuf.at[pl.ds(0, rem)],
                            y_sp.at[pl.ds(base_row + full * CH, rem)])
        plsc.subcore_barrier()

        def body(i, _):
            base = (wid * chunks_per_tile + i) * CH
            pltpu.sync_copy(row_hbm.at[pl.ds(base, CH)], rowbuf)
            pltpu.sync_copy(col_hbm.at[pl.ds(base, CH)], colbuf)
            pltpu.async_copy(xp_hbm.at[rowbuf], gbuf, sem).wait()
            pltpu.sync_copy(gbuf, y_sp.at[colbuf], add=True)
            return 0
        lax.fori_loop(0, chunks_per_tile, body, 0)

        plsc.subcore_barrier()
        pltpu.sync_copy(
            y_sp.at[pl.ds(base_row, rows_per_tile)],
            p_out.at[cidx, pl.ds(base_row, rows_per_tile)],
        )

    return agg_kernel


def _prep_body(degp_ref, x_ref, xp_ref, dinv_ref):
    p = degp_ref[...]
    deg = p[0] + p[1] + 1.0
    dinv = lax.rsqrt(deg)
    dinv_ref[...] = dinv
    xp_ref[...] = x_ref[...] * dinv


def _prep_call(degp3, x, blk):
    n, d = x.shape
    grid = n // blk
    return pl.pallas_call(
        _prep_body,
        grid=(grid,),
        in_specs=[
            pl.BlockSpec((NC, blk, 1), lambda i: (0, i, 0)),
            pl.BlockSpec((blk, d), lambda i: (i, 0)),
        ],
        out_specs=[
            pl.BlockSpec((blk, d), lambda i: (i, 0)),
            pl.BlockSpec((blk, 1), lambda i: (i, 0)),
        ],
        out_shape=[
            jax.ShapeDtypeStruct((n, d), jnp.float32),
            jax.ShapeDtypeStruct((n, 1), jnp.float32),
        ],
    )(degp3, x)


def _dense_body(p_ref, xp_ref, dinv_ref, wh_ref, bh_ref, wt_ref, bt_ref,
                wf_ref, bf_ref, wc_ref, bc_ref, out_ref):
    p = p_ref[...]
    y = dinv_ref[...] * (p[0] + p[1] + xp_ref[...])
    hh = jnp.maximum(jnp.dot(y, wh_ref[...], preferred_element_type=jnp.float32)
                     + bh_ref[...], 0.0)
    ht = jnp.maximum(jnp.dot(y, wt_ref[...], preferred_element_type=jnp.float32)
                     + bt_ref[...], 0.0)
    hf = 0.5 * hh + 0.5 * ht
    h = jnp.maximum(jnp.dot(hf, wf_ref[...], preferred_element_type=jnp.float32)
                    + bf_ref[...], 0.0)
    out_ref[...] = jnp.dot(h, wc_ref[...], preferred_element_type=jnp.float32) \
        + bc_ref[...]


def _dense_call(p, xp, dinv, wh, bh, wt, bt, wf, bf, wc_pad, bc_pad, blk):
    n, d = xp.shape
    hid = wh.shape[1]
    f = wf.shape[1]
    cpad = wc_pad.shape[1]
    grid = n // blk

    def full(shape):
        return pl.BlockSpec(shape, lambda i: tuple(0 for _ in shape))

    return pl.pallas_call(
        _dense_body,
        grid=(grid,),
        in_specs=[
            pl.BlockSpec((NC, blk, d), lambda i: (0, i, 0)),
            pl.BlockSpec((blk, d), lambda i: (i, 0)),
            pl.BlockSpec((blk, 1), lambda i: (i, 0)),
            full((d, hid)), full((1, hid)),
            full((d, hid)), full((1, hid)),
            full((hid, f)), full((1, f)),
            full((f, cpad)), full((1, cpad)),
        ],
        out_specs=pl.BlockSpec((blk, cpad), lambda i: (i, 0)),
        out_shape=jax.ShapeDtypeStruct((n, cpad), jnp.float32),
    )(p, xp, dinv, wh, bh, wt, bt, wf, bf, wc_pad, bc_pad)


def kernel(x, edge_index, h_node, W_homo, b_homo, W_hetero, b_hetero,
           W_fus, b_fus, W_cls, b_cls):
    del h_node
    n, d = x.shape
    e = edge_index.shape[1]

    n_pad = ((n + 1 + NS * 8 - 1) // (NS * 8)) * (NS * 8)
    ndeg = ((n + 1 + NS * L - 1) // (NS * L)) * (NS * L)
    e_pad = ((e + NW * CH - 1) // (NW * CH)) * (NW * CH)

    row = edge_index[0].astype(jnp.int32)
    col = edge_index[1].astype(jnp.int32)
    pad = jnp.full((e_pad - e,), n, jnp.int32)
    row_pad = jnp.concatenate([row, pad])
    col_pad = jnp.concatenate([col, pad])

    degp = _make_deg_kernel(e_pad, ndeg)(col_pad)
    degp3 = degp[:, :n].reshape(NC, n, 1)

    blk = 2000
    xprime, dinv = _prep_call(degp3, x, blk)

    xp_pad = jnp.pad(xprime, ((0, n_pad - n), (0, 0)))
    p = _make_agg_kernel(e_pad, n_pad, d)(xp_pad, row_pad, col_pad)

    bh = b_homo.reshape(1, -1)
    bt = b_hetero.reshape(1, -1)
    bf = b_fus.reshape(1, -1)
    ncls = W_cls.shape[1]
    wc_pad = jnp.pad(W_cls, ((0, 0), (0, 128 - ncls)))
    bc_pad = jnp.pad(b_cls.reshape(1, -1), ((0, 0), (0, 128 - ncls)))

    logits128 = _dense_call(p, xprime, dinv, W_homo, bh, W_hetero, bt,
                            W_fus, bf, wc_pad, bc_pad, blk)
    return logits128[:, :ncls]
